# baseline (speedup 1.0000x reference)
import functools

import jax
import jax.numpy as jnp
from jax import lax
from jax.experimental import pallas as pl
from jax.experimental.pallas import tpu as pltpu
from jax.experimental.pallas import tpu_sc as plsc

_NC = 2
_NS = 16
_NW = _NC * _NS
_K = 128
_SPLIT0 = 0.27

def _dot(a, b):
    return jnp.dot(a, b, preferred_element_type=jnp.float32)


def _ini_body(x_ref, w1_ref, b1_ref, w2_ref, b2_ref, out_ref):
    h = jnp.maximum(_dot(x_ref[...], w1_ref[...]) + b1_ref[...], 0.0)
    out_ref[...] = _dot(h, w2_ref[...]) + b2_ref[...]


def _layer_body(n_nodes, h_ref, agg_ref, w1_ref, b1_ref, w2_ref, b2_ref,
                gamma_ref, beta_ref, acc_ref, lw_ref, hout_ref, accout_ref):
    z = h_ref[...] + agg_ref[0, :n_nodes, :] + agg_ref[1, :n_nodes, :]
    z = jnp.maximum(_dot(z, w1_ref[...]) + b1_ref[...], 0.0)
    z = _dot(z, w2_ref[...]) + b2_ref[...]
    z = jnp.maximum(z, 0.0)
    mean = jnp.mean(z, axis=0, keepdims=True)
    zc = z - mean
    var = jnp.mean(zc * zc, axis=0, keepdims=True)
    hn = (z - mean) / jnp.sqrt(var + 1e-5) * gamma_ref[...] + beta_ref[...]
    hout_ref[...] = hn
    accout_ref[...] = acc_ref[...] + lw_ref[0, 0] * hn


def _pool_body(n_graphs, acc_ref, batch_ref, lb_ref, out_ref):
    n = acc_ref.shape[0]
    gids = lax.broadcasted_iota(jnp.int32, (n_graphs, n), 0)
    onehot = (gids == batch_ref[...]).astype(jnp.float32)
    out_ref[...] = jnp.dot(onehot, acc_ref[...] + lb_ref[0, 0],
                           preferred_element_type=jnp.float32,
                           precision=lax.Precision.HIGHEST)


@functools.partial(jax.jit, static_argnums=(4, 5, 6))
def _sc_scatter_add(h, src_t, dst_t, zrows, n_pad, c0, c1):
    dim = h.shape[1]
    rows_per_tile = n_pad // _NS
    c_max = max(c0, c1)
    mesh = plsc.VectorSubcoreMesh(core_axis_name="c", subcore_axis_name="s")

    @functools.partial(
        pl.kernel,
        out_type=jax.ShapeDtypeStruct((_NC, n_pad, dim), jnp.float32),
        mesh=mesh,
        scratch_types=[
            pltpu.VMEM((c_max, _K), jnp.int32),
            pltpu.VMEM((c_max, _K), jnp.int32),
            pltpu.VMEM((_K, dim), jnp.float32),
            pltpu.VMEM_SHARED((n_pad, dim), jnp.float32),
            pltpu.SemaphoreType.DMA,
        ],
    )
    def scatter_kernel(h_hbm, src_hbm, dst_hbm, z_hbm, out_hbm,
                       src_v, dst_v, buf, agg_sh, sem):
        cid = lax.axis_index("c")
        sid = lax.axis_index("s")
        wid = cid * _NS + sid
        pltpu.sync_copy(src_hbm.at[wid], src_v)
        pltpu.sync_copy(dst_hbm.at[wid], dst_v)
        pltpu.sync_copy(z_hbm, agg_sh.at[pl.ds(sid * rows_per_tile, rows_per_tile)])
        plsc.subcore_barrier()

        n_chunks = jnp.where(cid == 0, c0, c1)

        @pl.loop(0, n_chunks)
        def _(j):
            pltpu.async_copy(h_hbm.at[src_v.at[j]], buf, sem).wait()
            pltpu.sync_copy(buf, agg_sh.at[dst_v.at[j]], add=True)

        plsc.subcore_barrier()
        pltpu.sync_copy(
            agg_sh.at[pl.ds(sid * rows_per_tile, rows_per_tile)],
            out_hbm.at[cid, pl.ds(sid * rows_per_tile, rows_per_tile)])

    return scatter_kernel(h, src_t, dst_t, zrows)


def kernel(x, edge_index, batch, percent, W_e1, b_e1, W_e2, b_e2,
           Wg1, bg1, Wg2, bg2, gamma, beta, layer_w, layer_b):
    n, df = x.shape
    dim = W_e1.shape[1]
    e = edge_index.shape[1]
    n_layers = Wg1.shape[0]
    n_graphs = 64

    total_chunks = -(-e // _K)
    c0 = max(1, round(total_chunks * _SPLIT0 / _NS))
    c1 = max(1, -(-(e - _NS * c0 * _K) // (_NS * _K)))
    c_max = max(c0, c1)
    e_pad = _NS * (c0 + c1) * _K
    n_pad = -(-(n + 1) // (_NS * 8)) * (_NS * 8)

    src = edge_index[0]
    dst = edge_index[1]
    pad = e_pad - e

    def tile_layout(ids, fill):
        idsp = jnp.concatenate([ids, jnp.full((pad,), fill, jnp.int32)])
        split = _NS * c0 * _K
        t0 = idsp[:split].reshape(_NS, c0, _K)
        t0 = jnp.concatenate(
            [t0, jnp.full((_NS, c_max - c0, _K), fill, jnp.int32)], axis=1)
        t1 = idsp[split:].reshape(_NS, c1, _K)
        t1 = jnp.concatenate(
            [t1, jnp.full((_NS, c_max - c1, _K), fill, jnp.int32)], axis=1)
        return jnp.concatenate([t0, t1], axis=0)

    src_t = tile_layout(src, 0)
    dst_t = tile_layout(dst, n)
    zrows = jnp.zeros((n_pad // _NS, dim), jnp.float32)

    f32 = jnp.float32
    h = pl.pallas_call(
        _ini_body,
        out_shape=jax.ShapeDtypeStruct((n, dim), f32),
    )(x, W_e1, b_e1.reshape(1, dim), W_e2, b_e2.reshape(1, dim))

    acc = jnp.zeros((n, dim), f32)
    layer_fn = pl.pallas_call(
        functools.partial(_layer_body, n),
        out_shape=(jax.ShapeDtypeStruct((n, dim), f32),
                   jax.ShapeDtypeStruct((n, dim), f32)),
    )
    for i in range(n_layers):
        agg = _sc_scatter_add(h, src_t, dst_t, zrows, n_pad, c0, c1)
        h, acc = layer_fn(h, agg, Wg1[i], bg1[i].reshape(1, dim),
                          Wg2[i], bg2[i].reshape(1, dim),
                          gamma[i].reshape(1, dim), beta[i].reshape(1, dim),
                          acc, layer_w[i].reshape(1, 1))

    out = pl.pallas_call(
        functools.partial(_pool_body, n_graphs),
        out_shape=jax.ShapeDtypeStruct((n_graphs, dim), f32),
    )(acc, batch.reshape(1, n), layer_b.reshape(1, 1))
    return out


# speedup vs baseline: 1.1215x; 1.1215x over previous
import functools

import jax
import jax.numpy as jnp
from jax import lax
from jax.experimental import pallas as pl
from jax.experimental.pallas import tpu as pltpu
from jax.experimental.pallas import tpu_sc as plsc

_NC = 2
_NS = 16
_NW = _NC * _NS
_K = 128
_SPLIT0 = 0.40

def _dot(a, b):
    return jnp.dot(a, b, preferred_element_type=jnp.float32)


def _ini_body(x_ref, w1_ref, b1_ref, w2_ref, b2_ref, out_ref):
    h = jnp.maximum(_dot(x_ref[...], w1_ref[...]) + b1_ref[...], 0.0)
    out_ref[...] = _dot(h, w2_ref[...]) + b2_ref[...]


def _layer_body(n_nodes, h_ref, agg_ref, w1_ref, b1_ref, w2_ref, b2_ref,
                gamma_ref, beta_ref, acc_ref, lw_ref, hout_ref, accout_ref):
    z = h_ref[...] + agg_ref[0, :n_nodes, :] + agg_ref[1, :n_nodes, :]
    z = jnp.maximum(_dot(z, w1_ref[...]) + b1_ref[...], 0.0)
    z = _dot(z, w2_ref[...]) + b2_ref[...]
    z = jnp.maximum(z, 0.0)
    mean = jnp.mean(z, axis=0, keepdims=True)
    zc = z - mean
    var = jnp.mean(zc * zc, axis=0, keepdims=True)
    hn = (z - mean) / jnp.sqrt(var + 1e-5) * gamma_ref[...] + beta_ref[...]
    hout_ref[...] = hn
    accout_ref[...] = acc_ref[...] + lw_ref[0, 0] * hn


def _pool_body(n_graphs, acc_ref, batch_ref, lb_ref, out_ref):
    n = acc_ref.shape[0]
    gids = lax.broadcasted_iota(jnp.int32, (n_graphs, n), 0)
    onehot = (gids == batch_ref[...]).astype(jnp.float32)
    out_ref[...] = jnp.dot(onehot, acc_ref[...] + lb_ref[0, 0],
                           preferred_element_type=jnp.float32,
                           precision=lax.Precision.HIGHEST)


@functools.partial(jax.jit, static_argnums=(4, 5, 6))
def _sc_scatter_add(h, src_t, dst_t, zrows, n_pad, c0, c1):
    dim = h.shape[1]
    rows_per_tile = n_pad // _NS
    c_max = max(c0, c1)
    mesh = plsc.VectorSubcoreMesh(core_axis_name="c", subcore_axis_name="s")

    @functools.partial(
        pl.kernel,
        out_type=jax.ShapeDtypeStruct((_NC, n_pad, dim), jnp.float32),
        mesh=mesh,
        scratch_types=[
            pltpu.VMEM((c_max, _K), jnp.int32),
            pltpu.VMEM((c_max, _K), jnp.int32),
            pltpu.VMEM((_K, dim), jnp.float32),
            pltpu.VMEM_SHARED((n_pad, dim), jnp.float32),
            pltpu.SemaphoreType.DMA,
        ],
    )
    def scatter_kernel(h_hbm, src_hbm, dst_hbm, z_hbm, out_hbm,
                       src_v, dst_v, buf, agg_sh, sem):
        cid = lax.axis_index("c")
        sid = lax.axis_index("s")
        wid = cid * _NS + sid
        pltpu.sync_copy(src_hbm.at[wid], src_v)
        pltpu.sync_copy(dst_hbm.at[wid], dst_v)
        pltpu.sync_copy(z_hbm, agg_sh.at[pl.ds(sid * rows_per_tile, rows_per_tile)])
        plsc.subcore_barrier()

        n_chunks = jnp.where(cid == 0, c0, c1)

        @pl.loop(0, n_chunks)
        def _(j):
            pltpu.async_copy(h_hbm.at[src_v.at[j]], buf, sem).wait()
            pltpu.sync_copy(buf, agg_sh.at[dst_v.at[j]], add=True)

        plsc.subcore_barrier()
        pltpu.sync_copy(
            agg_sh.at[pl.ds(sid * rows_per_tile, rows_per_tile)],
            out_hbm.at[cid, pl.ds(sid * rows_per_tile, rows_per_tile)])

    return scatter_kernel(h, src_t, dst_t, zrows)


def kernel(x, edge_index, batch, percent, W_e1, b_e1, W_e2, b_e2,
           Wg1, bg1, Wg2, bg2, gamma, beta, layer_w, layer_b):
    n, df = x.shape
    dim = W_e1.shape[1]
    e = edge_index.shape[1]
    n_layers = Wg1.shape[0]
    n_graphs = 64

    total_chunks = -(-e // _K)
    c0 = max(1, round(total_chunks * _SPLIT0 / _NS))
    c1 = max(1, -(-(e - _NS * c0 * _K) // (_NS * _K)))
    c_max = max(c0, c1)
    e_pad = _NS * (c0 + c1) * _K
    n_pad = -(-(n + 1) // (_NS * 8)) * (_NS * 8)

    src = edge_index[0]
    dst = edge_index[1]
    pad = e_pad - e

    def tile_layout(ids, fill):
        idsp = jnp.concatenate([ids, jnp.full((pad,), fill, jnp.int32)])
        split = _NS * c0 * _K
        t0 = idsp[:split].reshape(_NS, c0, _K)
        t0 = jnp.concatenate(
            [t0, jnp.full((_NS, c_max - c0, _K), fill, jnp.int32)], axis=1)
        t1 = idsp[split:].reshape(_NS, c1, _K)
        t1 = jnp.concatenate(
            [t1, jnp.full((_NS, c_max - c1, _K), fill, jnp.int32)], axis=1)
        return jnp.concatenate([t0, t1], axis=0)

    src_t = tile_layout(src, 0)
    dst_t = tile_layout(dst, n)
    zrows = jnp.zeros((n_pad // _NS, dim), jnp.float32)

    f32 = jnp.float32
    h = pl.pallas_call(
        _ini_body,
        out_shape=jax.ShapeDtypeStruct((n, dim), f32),
    )(x, W_e1, b_e1.reshape(1, dim), W_e2, b_e2.reshape(1, dim))

    acc = jnp.zeros((n, dim), f32)
    layer_fn = pl.pallas_call(
        functools.partial(_layer_body, n),
        out_shape=(jax.ShapeDtypeStruct((n, dim), f32),
                   jax.ShapeDtypeStruct((n, dim), f32)),
    )
    for i in range(n_layers):
        agg = _sc_scatter_add(h, src_t, dst_t, zrows, n_pad, c0, c1)
        h, acc = layer_fn(h, agg, Wg1[i], bg1[i].reshape(1, dim),
                          Wg2[i], bg2[i].reshape(1, dim),
                          gamma[i].reshape(1, dim), beta[i].reshape(1, dim),
                          acc, layer_w[i].reshape(1, 1))

    out = pl.pallas_call(
        functools.partial(_pool_body, n_graphs),
        out_shape=jax.ShapeDtypeStruct((n_graphs, dim), f32),
    )(acc, batch.reshape(1, n), layer_b.reshape(1, 1))
    return out


# speedup vs baseline: 1.1359x; 1.0129x over previous
import functools

import jax
import jax.numpy as jnp
from jax import lax
from jax.experimental import pallas as pl
from jax.experimental.pallas import tpu as pltpu
from jax.experimental.pallas import tpu_sc as plsc

_NC = 2
_NS = 16
_NW = _NC * _NS
_K = 128
_SPLIT0 = 0.35

def _dot(a, b):
    return jnp.dot(a, b, preferred_element_type=jnp.float32)


def _ini_body(x_ref, w1_ref, b1_ref, w2_ref, b2_ref, out_ref):
    h = jnp.maximum(_dot(x_ref[...], w1_ref[...]) + b1_ref[...], 0.0)
    out_ref[...] = _dot(h, w2_ref[...]) + b2_ref[...]


def _layer_body(n_nodes, h_ref, agg_ref, w1_ref, b1_ref, w2_ref, b2_ref,
                gamma_ref, beta_ref, acc_ref, lw_ref, hout_ref, accout_ref):
    z = h_ref[...] + agg_ref[0, :n_nodes, :] + agg_ref[1, :n_nodes, :]
    z = jnp.maximum(_dot(z, w1_ref[...]) + b1_ref[...], 0.0)
    z = _dot(z, w2_ref[...]) + b2_ref[...]
    z = jnp.maximum(z, 0.0)
    mean = jnp.mean(z, axis=0, keepdims=True)
    zc = z - mean
    var = jnp.mean(zc * zc, axis=0, keepdims=True)
    hn = (z - mean) / jnp.sqrt(var + 1e-5) * gamma_ref[...] + beta_ref[...]
    hout_ref[...] = hn
    accout_ref[...] = acc_ref[...] + lw_ref[0, 0] * hn


def _pool_body(n_graphs, acc_ref, batch_ref, lb_ref, out_ref):
    n = acc_ref.shape[0]
    gids = lax.broadcasted_iota(jnp.int32, (n_graphs, n), 0)
    onehot = (gids == batch_ref[...]).astype(jnp.float32)
    out_ref[...] = jnp.dot(onehot, acc_ref[...] + lb_ref[0, 0],
                           preferred_element_type=jnp.float32,
                           precision=lax.Precision.HIGHEST)


@functools.partial(jax.jit, static_argnums=(4, 5, 6))
def _sc_scatter_add(h, src_t, dst_t, zrows, n_pad, c0, c1):
    dim = h.shape[1]
    rows_per_tile = n_pad // _NS
    c_max = max(c0, c1)
    mesh = plsc.VectorSubcoreMesh(core_axis_name="c", subcore_axis_name="s")

    @functools.partial(
        pl.kernel,
        out_type=jax.ShapeDtypeStruct((_NC, n_pad, dim), jnp.float32),
        mesh=mesh,
        scratch_types=[
            pltpu.VMEM((c_max, _K), jnp.int32),
            pltpu.VMEM((c_max, _K), jnp.int32),
            pltpu.VMEM((_K, dim), jnp.float32),
            pltpu.VMEM_SHARED((n_pad, dim), jnp.float32),
            pltpu.SemaphoreType.DMA,
        ],
    )
    def scatter_kernel(h_hbm, src_hbm, dst_hbm, z_hbm, out_hbm,
                       src_v, dst_v, buf, agg_sh, sem):
        cid = lax.axis_index("c")
        sid = lax.axis_index("s")
        wid = cid * _NS + sid
        pltpu.sync_copy(src_hbm.at[wid], src_v)
        pltpu.sync_copy(dst_hbm.at[wid], dst_v)
        pltpu.sync_copy(z_hbm, agg_sh.at[pl.ds(sid * rows_per_tile, rows_per_tile)])
        plsc.subcore_barrier()

        n_chunks = jnp.where(cid == 0, c0, c1)

        @pl.loop(0, n_chunks)
        def _(j):
            pltpu.async_copy(h_hbm.at[src_v.at[j]], buf, sem).wait()
            pltpu.sync_copy(buf, agg_sh.at[dst_v.at[j]], add=True)

        plsc.subcore_barrier()
        pltpu.sync_copy(
            agg_sh.at[pl.ds(sid * rows_per_tile, rows_per_tile)],
            out_hbm.at[cid, pl.ds(sid * rows_per_tile, rows_per_tile)])

    return scatter_kernel(h, src_t, dst_t, zrows)


def kernel(x, edge_index, batch, percent, W_e1, b_e1, W_e2, b_e2,
           Wg1, bg1, Wg2, bg2, gamma, beta, layer_w, layer_b):
    n, df = x.shape
    dim = W_e1.shape[1]
    e = edge_index.shape[1]
    n_layers = Wg1.shape[0]
    n_graphs = 64

    total_chunks = -(-e // _K)
    c0 = max(1, round(total_chunks * _SPLIT0 / _NS))
    c1 = max(1, -(-(e - _NS * c0 * _K) // (_NS * _K)))
    c_max = max(c0, c1)
    e_pad = _NS * (c0 + c1) * _K
    n_pad = -(-(n + 1) // (_NS * 8)) * (_NS * 8)

    src = edge_index[0]
    dst = edge_index[1]
    pad = e_pad - e

    def tile_layout(ids, fill):
        idsp = jnp.concatenate([ids, jnp.full((pad,), fill, jnp.int32)])
        split = _NS * c0 * _K
        t0 = idsp[:split].reshape(_NS, c0, _K)
        t0 = jnp.concatenate(
            [t0, jnp.full((_NS, c_max - c0, _K), fill, jnp.int32)], axis=1)
        t1 = idsp[split:].reshape(_NS, c1, _K)
        t1 = jnp.concatenate(
            [t1, jnp.full((_NS, c_max - c1, _K), fill, jnp.int32)], axis=1)
        return jnp.concatenate([t0, t1], axis=0)

    src_t = tile_layout(src, 0)
    dst_t = tile_layout(dst, n)
    zrows = jnp.zeros((n_pad // _NS, dim), jnp.float32)

    f32 = jnp.float32
    h = pl.pallas_call(
        _ini_body,
        out_shape=jax.ShapeDtypeStruct((n, dim), f32),
    )(x, W_e1, b_e1.reshape(1, dim), W_e2, b_e2.reshape(1, dim))

    acc = jnp.zeros((n, dim), f32)
    layer_fn = pl.pallas_call(
        functools.partial(_layer_body, n),
        out_shape=(jax.ShapeDtypeStruct((n, dim), f32),
                   jax.ShapeDtypeStruct((n, dim), f32)),
    )
    for i in range(n_layers):
        agg = _sc_scatter_add(h, src_t, dst_t, zrows, n_pad, c0, c1)
        h, acc = layer_fn(h, agg, Wg1[i], bg1[i].reshape(1, dim),
                          Wg2[i], bg2[i].reshape(1, dim),
                          gamma[i].reshape(1, dim), beta[i].reshape(1, dim),
                          acc, layer_w[i].reshape(1, 1))

    out = pl.pallas_call(
        functools.partial(_pool_body, n_graphs),
        out_shape=jax.ShapeDtypeStruct((n_graphs, dim), f32),
    )(acc, batch.reshape(1, n), layer_b.reshape(1, 1))
    return out
